# baseline (speedup 1.0000x reference)
import functools
import math

import jax
import jax.numpy as jnp
from jax import lax
from jax.experimental import pallas as pl
from jax.experimental.pallas import tpu as pltpu
from jax.experimental.pallas import tpu_sc as plsc

D_EMB = 128
SEQ = 200
BATCH = 1024
NUM_CORES = 2
NUM_SUBCORES = 16
NW = NUM_CORES * NUM_SUBCORES
ROWS_PER_W = BATCH // NW
LANES = 16
SCALE = math.sqrt(float(D_EMB))
GATHER_SPLITS = ((0, 128), (128, 72))
NBUF = 3


def kernel(x, table, pe):
    B, S = x.shape
    V, D = table.shape
    assert (B, S, D) == (BATCH, SEQ, D_EMB)
    xf = x.reshape(B * S).astype(jnp.int32)
    pe_s = pe[:S]

    mesh = plsc.VectorSubcoreMesh(core_axis_name="c", subcore_axis_name="s")

    @functools.partial(
        pl.kernel,
        out_type=jax.ShapeDtypeStruct((B * S, D), jnp.float32),
        mesh=mesh,
        scratch_types=[
            pltpu.VMEM((ROWS_PER_W * SEQ,), jnp.int32),
            pltpu.VMEM((SEQ, D_EMB), jnp.float32),
            pltpu.VMEM((SEQ, D_EMB), jnp.float32),
            pltpu.VMEM((SEQ, D_EMB), jnp.float32),
            pltpu.VMEM((SEQ, D_EMB), jnp.float32),
            pltpu.SemaphoreType.DMA,
            pltpu.SemaphoreType.DMA,
            pltpu.SemaphoreType.DMA,
            pltpu.SemaphoreType.DMA,
            pltpu.SemaphoreType.DMA,
            pltpu.SemaphoreType.DMA,
        ],
    )
    def emb_kernel(table_hbm, xf_hbm, pe_hbm, out_hbm, idx_v, pe_v,
                   rows0, rows1, rows2, g0, g1, g2, w0, w1, w2):
        wid = lax.axis_index("s") * NUM_CORES + lax.axis_index("c")
        rows = (rows0, rows1, rows2)
        gsem = (g0, g1, g2)
        wsem = (w0, w1, w2)

        pltpu.sync_copy(xf_hbm.at[pl.ds(wid * (ROWS_PER_W * SEQ), ROWS_PER_W * SEQ)],
                        idx_v)
        pltpu.sync_copy(pe_hbm, pe_v)

        def gather_copies(r, b):
            for off, win in GATHER_SPLITS:
                yield pltpu.make_async_copy(
                    table_hbm.at[idx_v.at[pl.ds(r * SEQ + off, win)]],
                    rows[b].at[pl.ds(off, win)],
                    gsem[b],
                )

        def gather_start(r, b):
            for c in gather_copies(r, b):
                c.start()

        def gather_wait(r, b):
            for c in gather_copies(r, b):
                c.wait()

        def write_start(r, b):
            pltpu.async_copy(
                rows[b], out_hbm.at[pl.ds((wid * ROWS_PER_W + r) * SEQ, SEQ)],
                wsem[b])

        def write_wait(b):
            pltpu.make_async_copy(
                rows[b], out_hbm.at[pl.ds(0, SEQ)], wsem[b]).wait()

        def compute(b):
            buf = rows[b]

            @plsc.parallel_loop(0, SEQ, unroll=2)
            def _tok(i):
                for c in range(D_EMB // LANES):
                    sl = pl.ds(c * LANES, LANES)
                    buf[i, sl] = buf[i, sl] * SCALE + pe_v[i, sl]

        def substep(r, b, prefetch_wait):
            nb = (b + 1) % NBUF

            @pl.when(r + 1 < ROWS_PER_W)
            def _():
                if prefetch_wait:
                    write_wait(nb)
                gather_start(r + 1, nb)

            gather_wait(r, b)
            compute(b)
            write_start(r, b)

        gather_start(0, 0)
        substep(0, 0, prefetch_wait=False)
        substep(1, 1, prefetch_wait=False)

        @pl.loop(0, (ROWS_PER_W - 2) // NBUF)
        def _grp(g):
            base = NBUF * g + 2
            substep(base, 2, prefetch_wait=True)
            substep(base + 1, 0, prefetch_wait=True)
            substep(base + 2, 1, prefetch_wait=True)

        write_wait(2)
        write_wait(0)
        write_wait(1)

    out = emb_kernel(table, xf, pe_s)
    return out.reshape(B, S, D)


# speedup vs baseline: 1.0063x; 1.0063x over previous
import functools
import math

import jax
import jax.numpy as jnp
from jax import lax
from jax.experimental import pallas as pl
from jax.experimental.pallas import tpu as pltpu
from jax.experimental.pallas import tpu_sc as plsc

D_EMB = 128
SEQ = 200
BATCH = 1024
NUM_CORES = 2
NUM_SUBCORES = 16
NW = NUM_CORES * NUM_SUBCORES
ROWS_PER_W = BATCH // NW
LANES = 16
SCALE = math.sqrt(float(D_EMB))
GATHER_SPLITS = ((0, 128), (128, 72))
NBUF = 3


def kernel(x, table, pe):
    B, S = x.shape
    V, D = table.shape
    assert (B, S, D) == (BATCH, SEQ, D_EMB)
    xf = x.reshape(B * S).astype(jnp.int32)
    pe_s = pe[:S]

    mesh = plsc.VectorSubcoreMesh(core_axis_name="c", subcore_axis_name="s")

    @functools.partial(
        pl.kernel,
        out_type=jax.ShapeDtypeStruct((B * S, D), jnp.float32),
        mesh=mesh,
        scratch_types=[
            pltpu.VMEM((ROWS_PER_W * SEQ,), jnp.int32),
            pltpu.VMEM((SEQ, D_EMB), jnp.float32),
            pltpu.VMEM((SEQ, D_EMB), jnp.float32),
            pltpu.VMEM((SEQ, D_EMB), jnp.float32),
            pltpu.VMEM((SEQ, D_EMB), jnp.float32),
            pltpu.SemaphoreType.DMA,
            pltpu.SemaphoreType.DMA,
            pltpu.SemaphoreType.DMA,
            pltpu.SemaphoreType.DMA,
            pltpu.SemaphoreType.DMA,
            pltpu.SemaphoreType.DMA,
        ],
    )
    def emb_kernel(table_hbm, xf_hbm, pe_hbm, out_hbm, idx_v, pe_v,
                   rows0, rows1, rows2, g0, g1, g2, w0, w1, w2):
        wid = lax.axis_index("s") * NUM_CORES + lax.axis_index("c")
        rows = (rows0, rows1, rows2)
        gsem = (g0, g1, g2)
        wsem = (w0, w1, w2)

        pltpu.sync_copy(xf_hbm.at[pl.ds(wid * (ROWS_PER_W * SEQ), ROWS_PER_W * SEQ)],
                        idx_v)
        pltpu.sync_copy(pe_hbm, pe_v)

        def gather_copies(r, b):
            for off, win in GATHER_SPLITS:
                yield pltpu.make_async_copy(
                    table_hbm.at[idx_v.at[pl.ds(r * SEQ + off, win)]],
                    rows[b].at[pl.ds(off, win)],
                    gsem[b],
                )

        def gather_start(r, b):
            for c in gather_copies(r, b):
                c.start()

        def gather_wait(r, b):
            for c in gather_copies(r, b):
                c.wait()

        def write_start(r, b):
            pltpu.async_copy(
                rows[b], out_hbm.at[pl.ds((wid * ROWS_PER_W + r) * SEQ, SEQ)],
                wsem[b])

        def write_wait(b):
            pltpu.make_async_copy(
                rows[b], out_hbm.at[pl.ds(0, SEQ)], wsem[b]).wait()

        def compute(b):
            buf = rows[b]

            @pl.loop(0, SEQ)
            def _tok(i):
                for c in range(D_EMB // LANES):
                    sl = pl.ds(c * LANES, LANES)
                    buf[i, sl] = buf[i, sl] * SCALE + pe_v[i, sl]

        def substep(r, b, prefetch_wait):
            nb = (b + 1) % NBUF

            @pl.when(r + 1 < ROWS_PER_W)
            def _():
                if prefetch_wait:
                    write_wait(nb)
                gather_start(r + 1, nb)

            gather_wait(r, b)
            compute(b)
            write_start(r, b)

        gather_start(0, 0)
        substep(0, 0, prefetch_wait=False)
        substep(1, 1, prefetch_wait=False)

        @pl.loop(0, (ROWS_PER_W - 2) // NBUF)
        def _grp(g):
            base = NBUF * g + 2
            substep(base, 2, prefetch_wait=True)
            substep(base + 1, 0, prefetch_wait=True)
            substep(base + 2, 1, prefetch_wait=True)

        write_wait(2)
        write_wait(0)
        write_wait(1)

    out = emb_kernel(table, xf, pe_s)
    return out.reshape(B, S, D)
